# Initial kernel scaffold; baseline (speedup 1.0000x reference)
import jax
import jax.numpy as jnp
from jax.experimental import pallas as pl


def kernel(x, edge_index, W_fwd_l, b_fwd_l, W_fwd_r, W_bwd_l, b_bwd_l, W_bwd_r, gamma, beta):
    raise NotImplementedError("write your pallas kernel here")


# speedup vs baseline: 3.9295x; 3.9295x over previous
import jax
import jax.numpy as jnp
from jax import lax
from jax.experimental import pallas as pl
from jax.experimental.pallas import tpu as pltpu
from jax.experimental.pallas import tpu_sc as plsc

_N = 10000
_H = 64
_EPS = 1e-5

_NC = 2
_NS = 16
_CH = 128
_L = 16

_KBLK = 32
_N_PAD = 10240
_RPT = _N_PAD // _NS
_RHALF = _RPT // 2


def _stage_a(x_ref, wl_ref, wr_ref, t_ref, yr_ref):
    xv = x_ref[...]
    hi = jax.lax.Precision.HIGHEST
    t_ref[...] = jnp.dot(xv, wl_ref[...].T, precision=hi)
    yr_ref[...] = jnp.dot(xv, wr_ref[...].T, precision=hi)


def _stage_b_body(nblk):
    def body(t_hbm, gidx_hbm, sidx_hbm, z_hbm,
             acc_hbm, cnt_hbm,
             acc_sh, gidx_v, sidx_v, g_v, hist_v, sem):
        c = lax.axis_index("c")
        s = lax.axis_index("s")
        row0 = s * _RPT
        for q in range(2):
            pltpu.sync_copy(z_hbm, acc_sh.at[pl.ds(row0 + q * _RHALF, _RHALF)])
        zeros16 = jnp.zeros((_L,), jnp.float32)

        def zb(i, car):
            hist_v[pl.ds(i * _L, _L)] = zeros16
            return car

        lax.fori_loop(0, _N_PAD // _L, zb, 0)
        plsc.subcore_barrier()

        ones16 = jnp.ones((_L,), jnp.float32)
        cpb = _CH // _L

        def blk(b, car):
            pltpu.sync_copy(gidx_hbm.at[c, s, pl.ds(b * _KBLK, _KBLK)], gidx_v)
            pltpu.sync_copy(sidx_hbm.at[c, s, pl.ds(b * _KBLK, _KBLK)], sidx_v)

            def step(j, car2):
                pltpu.async_copy(t_hbm.at[gidx_v.at[j]], g_v, sem).wait()
                pltpu.sync_copy(g_v, acc_sh.at[sidx_v.at[j]], add=True)
                return car2

            lax.fori_loop(0, _KBLK, step, 0)

            def hstep(i, car2):
                idx = sidx_v[i // cpb, pl.ds((i % cpb) * _L, _L)]
                plsc.addupdate_scatter(hist_v, [idx], ones16)
                return car2

            lax.fori_loop(0, _KBLK * cpb, hstep, 0)
            return car

        lax.fori_loop(0, nblk, blk, 0)
        plsc.subcore_barrier()
        pltpu.sync_copy(acc_sh.at[pl.ds(row0, _RPT)],
                        acc_hbm.at[c, pl.ds(row0, _RPT)])
        pltpu.sync_copy(hist_v, cnt_hbm.at[c * _NS + s])

    return body


def _stage_c(acc_ref, cnt_ref, yr_ref, bf_ref, bb_ref, g_ref, b_ref, out_ref):
    agg_f = acc_ref[0, :_N, :_H]
    agg_b = acc_ref[1, :_N, _H:]
    cnt_in = jnp.sum(cnt_ref[: _NS, :_N], axis=0)
    cnt_out = jnp.sum(cnt_ref[_NS:, :_N], axis=0)
    hf = agg_f / jnp.maximum(cnt_in, 1.0)[:, None] + bf_ref[...][None, :] \
        + yr_ref[:_N, :_H]
    hb = agg_b / jnp.maximum(cnt_out, 1.0)[:, None] + bb_ref[...][None, :] \
        + yr_ref[:_N, _H:]
    h = jnp.concatenate([hf, hb], axis=1)
    mean = jnp.mean(h, axis=0, keepdims=True)
    var = jnp.mean((h - mean) ** 2, axis=0, keepdims=True)
    hn = (h - mean) / jnp.sqrt(var + _EPS) * g_ref[...][None, :] + b_ref[...][None, :]
    out_ref[...] = jnp.maximum(hn, 0.0)


def kernel(x, edge_index, W_fwd_l, b_fwd_l, W_fwd_r, W_bwd_l, b_bwd_l,
           W_bwd_r, gamma, beta):
    n, d = x.shape
    e = edge_index.shape[1]
    K = (e + _NS * _CH - 1) // (_NS * _CH)
    K = ((K + _KBLK - 1) // _KBLK) * _KBLK
    nblk = K // _KBLK
    e_pad = _NS * _CH * K

    x_pad = jnp.zeros((_N_PAD, d), jnp.float32).at[:n].set(x)
    wl = jnp.concatenate([W_fwd_l, W_bwd_l], axis=0)
    wr = jnp.concatenate([W_fwd_r, W_bwd_r], axis=0)

    t, yr = pl.pallas_call(
        _stage_a,
        out_shape=[
            jax.ShapeDtypeStruct((_N_PAD, 2 * _H), jnp.float32),
            jax.ShapeDtypeStruct((_N_PAD, 2 * _H), jnp.float32),
        ],
    )(x_pad, wl, wr)

    pad = jnp.full((e_pad - e,), n, jnp.int32)
    src3 = jnp.concatenate([edge_index[0], pad]).reshape(_NS, K, _CH)
    dst3 = jnp.concatenate([edge_index[1], pad]).reshape(_NS, K, _CH)
    gidx = jnp.stack([src3, dst3])
    sidx = jnp.stack([dst3, src3])
    zrows = jnp.zeros((_RHALF, 2 * _H), jnp.float32)

    sc_kernel = pl.kernel(
        _stage_b_body(nblk),
        out_type=[
            jax.ShapeDtypeStruct((_NC, _N_PAD, 2 * _H), jnp.float32),
            jax.ShapeDtypeStruct((_NC * _NS, _N_PAD), jnp.float32),
        ],
        mesh=plsc.VectorSubcoreMesh(core_axis_name="c", subcore_axis_name="s"),
        compiler_params=pltpu.CompilerParams(needs_layout_passes=False),
        scratch_types=[
            pltpu.VMEM_SHARED((_N_PAD, 2 * _H), jnp.float32),
            pltpu.VMEM((_KBLK, _CH), jnp.int32),
            pltpu.VMEM((_KBLK, _CH), jnp.int32),
            pltpu.VMEM((_CH, 2 * _H), jnp.float32),
            pltpu.VMEM((_N_PAD,), jnp.float32),
            pltpu.SemaphoreType.DMA,
        ],
    )
    acc, cnt = sc_kernel(t, gidx, sidx, zrows)

    out = pl.pallas_call(
        _stage_c,
        out_shape=jax.ShapeDtypeStruct((_N, 2 * _H), jnp.float32),
    )(acc, cnt, yr, b_fwd_l, b_bwd_l, gamma, beta)
    return out
